# baseline (speedup 1.0000x reference)
import functools

import jax
import jax.numpy as jnp
from jax import lax
from jax.experimental import pallas as pl
from jax.experimental.pallas import tpu as pltpu
from jax.experimental.pallas import tpu_sc as plsc

N_NODES = 10000
HID = 128
N_EDGES = 320000

N_PAD = 10112
ROWS_PER_TILE = N_PAD // 16
CHUNK = 128
E_PAD = 327680
EDGES_PER_TILE = E_PAD // 16
CHUNKS_PER_TILE = EDGES_PER_TILE // CHUNK


def _pre_body(h_ref, w1a_ref, w1b_ref, b1_ref, w2_ref, b2_ref, iw_ref,
              ib_ref, a_ref, t_ref):
    hv = h_ref[...]
    b1 = b1_ref[...]
    w2 = w2_ref[...]
    b2 = b2_ref[...]
    iw = iw_ref[...]
    ib = ib_ref[...]
    fa = jnp.dot(jax.nn.relu(jnp.dot(hv, w1a_ref[...],
                                     preferred_element_type=jnp.float32) + b1),
                 w2, preferred_element_type=jnp.float32) + b2
    fb = jnp.dot(jax.nn.relu(jnp.dot(hv, w1b_ref[...],
                                     preferred_element_type=jnp.float32) + b1),
                 w2, preferred_element_type=jnp.float32) + b2
    sa = jax.nn.sigmoid(jnp.dot(fa, iw, preferred_element_type=jnp.float32) + ib)
    sb = jax.nn.sigmoid(jnp.dot(fb, iw, preferred_element_type=jnp.float32) + ib)
    a_ref[...] = fa * sa
    t_ref[...] = fb * sb


def _tc_pre(h2, w1a, w1b, b1, w2, b2, iw, ib, blk=1000):
    grid = N_NODES // blk
    full = lambda s: pl.BlockSpec(s, lambda i: (0,) * len(s))
    return pl.pallas_call(
        _pre_body,
        grid=(grid,),
        in_specs=[
            pl.BlockSpec((blk, HID), lambda i: (i, 0)),
            full((HID, HID)), full((HID, HID)), full((1, HID)),
            full((HID, HID)), full((1, HID)), full((HID, 1)), full((1, 1)),
        ],
        out_specs=[
            pl.BlockSpec((blk, HID), lambda i: (i, 0)),
            pl.BlockSpec((blk, HID), lambda i: (i, 0)),
        ],
        out_shape=[
            jax.ShapeDtypeStruct((N_NODES, HID), jnp.float32),
            jax.ShapeDtypeStruct((N_NODES, HID), jnp.float32),
        ],
    )(h2, w1a, w1b, b1, w2, b2, iw, ib)


NBUF = 2
PASS_CHUNKS = 32
N_CHUNKS = E_PAD // CHUNK
K0 = 96
K1 = 64
DEG_CHUNKS = N_CHUNKS // 16
DEG_RING = 8


def _sc_body(gidx_hbm, sidx_hbm, tabs_hbm, z2d_hbm, ones_hbm, acc_out,
             g_idx, s_idx, rows0, rows1, acc_sh, gs0, gs1, ss0, ss1):
    cid = lax.axis_index("c")
    sid = lax.axis_index("s")
    bufs = (rows0, rows1)
    gsems = (gs0, gs1)
    ssems = (ss0, ss1)
    myrows = pl.ds(sid * ROWS_PER_TILE, ROWS_PER_TILE)

    pltpu.sync_copy(z2d_hbm.at[myrows], acc_sh.at[myrows])
    plsc.subcore_barrier()

    @pl.when(cid == 1)
    def _():
      with jax.named_scope("deg_phase"):
        pltpu.sync_copy(ones_hbm, rows0)
        for p in range(DEG_CHUNKS // PASS_CHUNKS):
            pltpu.sync_copy(
                sidx_hbm.at[pl.ds(DEG_CHUNKS * sid + PASS_CHUNKS * p,
                                  PASS_CHUNKS)], s_idx)

            def dgroup(i, carry):
                descs = []
                for j in range(DEG_RING):
                    descs.append(pltpu.async_copy(
                        rows0, acc_sh.at[s_idx.at[DEG_RING * i + j]],
                        ssems[j % 2], add=True))
                for j in range(DEG_RING):
                    descs[j].wait()
                return carry

            lax.fori_loop(0, PASS_CHUNKS // DEG_RING, dgroup, 0)
        plsc.subcore_barrier()
        pltpu.sync_copy(acc_sh.at[myrows],
                        acc_out.at[pl.ds(N_PAD + sid * ROWS_PER_TILE,
                                         ROWS_PER_TILE)])
        pltpu.sync_copy(z2d_hbm.at[myrows], acc_sh.at[myrows])
        plsc.subcore_barrier()

    base = jnp.where(cid == 0, K0 * sid, 16 * K0 + K1 * sid)

    def gather(c, b):
        return pltpu.async_copy(tabs_hbm.at[g_idx.at[c]], bufs[b], gsems[b])

    def run_pass(p):
        row0 = base + PASS_CHUNKS * p
        pltpu.sync_copy(gidx_hbm.at[pl.ds(row0, PASS_CHUNKS)], g_idx)
        pltpu.sync_copy(sidx_hbm.at[pl.ds(row0, PASS_CHUNKS)], s_idx)
        for b in range(NBUF):
            gather(b, b)

        def group(i, carry):
            descs = []
            for b in range(NBUF):
                c = NBUF * i + b
                pltpu.make_async_copy(tabs_hbm.at[g_idx.at[c]], bufs[b],
                                      gsems[b]).wait()
                descs.append(pltpu.async_copy(
                    bufs[b], acc_sh.at[s_idx.at[c]], ssems[b], add=True))
            for b in range(NBUF):
                nxt = NBUF * i + b + NBUF

                @pl.when(nxt < PASS_CHUNKS)
                def _():
                    descs[b].wait()
                    gather(nxt, b)
            return carry

        lax.fori_loop(0, PASS_CHUNKS // NBUF, group, 0)
        for b in range(NBUF):
            pltpu.make_async_copy(
                bufs[b], acc_sh.at[s_idx.at[PASS_CHUNKS - NBUF + b]],
                ssems[b]).wait()

    with jax.named_scope("t_phase_shared"):
        for p in range(K1 // PASS_CHUNKS):
            run_pass(p)
    with jax.named_scope("t_phase_extra"):
        for p in range(K1 // PASS_CHUNKS, K0 // PASS_CHUNKS):

            @pl.when(cid == 0)
            def _():
                run_pass(p)

    plsc.subcore_barrier()
    pltpu.sync_copy(
        acc_sh.at[myrows],
        acc_out.at[pl.ds(2 * N_PAD * cid + sid * ROWS_PER_TILE,
                         ROWS_PER_TILE)])


def _sc_scatter(gidx, sidx, tabs, z2d, ones2d):
    mesh = plsc.VectorSubcoreMesh(core_axis_name="c", subcore_axis_name="s")
    f = pl.kernel(
        _sc_body,
        mesh=mesh,
        out_type=[
            jax.ShapeDtypeStruct((3 * N_PAD, HID), jnp.float32),
        ],
        scratch_types=[
            pltpu.VMEM((PASS_CHUNKS, CHUNK), jnp.int32),
            pltpu.VMEM((PASS_CHUNKS, CHUNK), jnp.int32),
            pltpu.VMEM((CHUNK, HID), jnp.float32),
            pltpu.VMEM((CHUNK, HID), jnp.float32),
            pltpu.VMEM_SHARED((N_PAD, HID), jnp.float32),
            pltpu.SemaphoreType.DMA,
            pltpu.SemaphoreType.DMA,
            pltpu.SemaphoreType.DMA,
            pltpu.SemaphoreType.DMA,
        ],
    )
    return f(gidx, sidx, tabs, z2d, ones2d)


def _post_body(h_ref, a_ref, accT0_ref, accT1_ref, deg_ref, nw1a_ref,
               nw1b_ref, nb1_ref, nw2_ref, nb2_ref, out_ref):
    hv = h_ref[...]
    nb1 = nb1_ref[...]
    nw2 = nw2_ref[...]
    nb2 = nb2_ref[...]
    mi1 = accT0_ref[...] + accT1_ref[...]
    mi0 = a_ref[...] * deg_ref[...][:, 0:1]

    def mlp(v, w1):
        return jnp.dot(jax.nn.relu(
            jnp.dot(v, w1, preferred_element_type=jnp.float32) + nb1),
            nw2, preferred_element_type=jnp.float32) + nb2

    u0 = hv + mlp(mi0, nw1a_ref[...])
    u1 = hv + mlp(mi1, nw1a_ref[...])
    u2 = hv + mlp(hv, nw1b_ref[...])
    out_ref[...] = jnp.concatenate(
        [u0[:, None, :], u1[:, None, :], u2[:, None, :]], axis=1)


def _tc_post(h2, a, accT0, accT1, deg, nw1a, nw1b, nb1, nw2, nb2, blk=1000):
    grid = N_NODES // blk
    full = lambda s: pl.BlockSpec(s, lambda i: (0,) * len(s))
    return pl.pallas_call(
        _post_body,
        grid=(grid,),
        in_specs=[
            pl.BlockSpec((blk, HID), lambda i: (i, 0)),
            pl.BlockSpec((blk, HID), lambda i: (i, 0)),
            pl.BlockSpec((blk, HID), lambda i: (i, 0)),
            pl.BlockSpec((blk, HID), lambda i: (i, 0)),
            pl.BlockSpec((blk, HID), lambda i: (i, 0)),
            full((HID, HID)), full((HID, HID)), full((1, HID)),
            full((HID, HID)), full((1, HID)),
        ],
        out_specs=pl.BlockSpec((blk, 3, HID), lambda i: (i, 0, 0)),
        out_shape=jax.ShapeDtypeStruct((N_NODES, 3, HID), jnp.float32),
    )(h2, a, accT0, accT1, deg, nw1a, nw1b, nb1, nw2, nb2)


def kernel(h, x, edge_index, edge_W1, edge_b1, edge_W2, edge_b2, inf_W,
           inf_b, node_W1, node_b1, node_W2, node_b2):
    h2 = h[:, 0, :]
    a_tab, t_tab = _tc_pre(
        h2, edge_W1[:HID], edge_W1[HID:], edge_b1[None, :], edge_W2,
        edge_b2[None, :], inf_W, inf_b[None, :])

    src = edge_index[0]
    dst = edge_index[1]
    pad = E_PAD - N_EDGES
    gidx = jnp.concatenate([src, jnp.zeros((pad,), jnp.int32)]).reshape(
        N_CHUNKS, CHUNK)
    sidx = jnp.concatenate(
        [dst, jnp.full((pad,), N_NODES, jnp.int32)]).reshape(
        N_CHUNKS, CHUNK)
    z2d = jnp.zeros((N_PAD, HID), jnp.float32)
    ones2d = jnp.ones((CHUNK, HID), jnp.float32)

    (acc_flat,) = _sc_scatter(gidx, sidx, t_tab, z2d, ones2d)
    accT0 = acc_flat[:N_NODES]
    deg = acc_flat[N_PAD:N_PAD + N_NODES]
    accT1 = acc_flat[2 * N_PAD:2 * N_PAD + N_NODES]

    h_new = _tc_post(h2, a_tab, accT0, accT1, deg, node_W1[:HID],
                     node_W1[HID:], node_b1[None, :], node_W2,
                     node_b2[None, :])
    return (h_new, x)


# speedup vs baseline: 1.0244x; 1.0244x over previous
import functools

import jax
import jax.numpy as jnp
from jax import lax
from jax.experimental import pallas as pl
from jax.experimental.pallas import tpu as pltpu
from jax.experimental.pallas import tpu_sc as plsc

N_NODES = 10000
HID = 128
N_EDGES = 320000

N_PAD = 10112
ROWS_PER_TILE = N_PAD // 16
CHUNK = 128
E_PAD = 327680
EDGES_PER_TILE = E_PAD // 16
CHUNKS_PER_TILE = EDGES_PER_TILE // CHUNK


def _pre_body(h_ref, w1a_ref, w1b_ref, b1_ref, w2_ref, b2_ref, iw_ref,
              ib_ref, a_ref, t_ref):
    hv = h_ref[...]
    b1 = b1_ref[...]
    w2 = w2_ref[...]
    b2 = b2_ref[...]
    iw = iw_ref[...]
    ib = ib_ref[...]
    fa = jnp.dot(jax.nn.relu(jnp.dot(hv, w1a_ref[...],
                                     preferred_element_type=jnp.float32) + b1),
                 w2, preferred_element_type=jnp.float32) + b2
    fb = jnp.dot(jax.nn.relu(jnp.dot(hv, w1b_ref[...],
                                     preferred_element_type=jnp.float32) + b1),
                 w2, preferred_element_type=jnp.float32) + b2
    sa = jax.nn.sigmoid(jnp.dot(fa, iw, preferred_element_type=jnp.float32) + ib)
    sb = jax.nn.sigmoid(jnp.dot(fb, iw, preferred_element_type=jnp.float32) + ib)
    a_ref[...] = fa * sa
    t_ref[...] = fb * sb


def _tc_pre(h2, w1a, w1b, b1, w2, b2, iw, ib, blk=1000):
    grid = N_NODES // blk
    full = lambda s: pl.BlockSpec(s, lambda i: (0,) * len(s))
    return pl.pallas_call(
        _pre_body,
        grid=(grid,),
        in_specs=[
            pl.BlockSpec((blk, HID), lambda i: (i, 0)),
            full((HID, HID)), full((HID, HID)), full((1, HID)),
            full((HID, HID)), full((1, HID)), full((HID, 1)), full((1, 1)),
        ],
        out_specs=[
            pl.BlockSpec((blk, HID), lambda i: (i, 0)),
            pl.BlockSpec((blk, HID), lambda i: (i, 0)),
        ],
        out_shape=[
            jax.ShapeDtypeStruct((N_NODES, HID), jnp.float32),
            jax.ShapeDtypeStruct((N_NODES, HID), jnp.float32),
        ],
    )(h2, w1a, w1b, b1, w2, b2, iw, ib)


NBUF = 2
PASS_CHUNKS = 40
N_CHUNKS = E_PAD // CHUNK
K0 = 120
K1 = 40
DEG_CHUNKS = N_CHUNKS // 16
DEG_RING = 8


def _sc_body(gidx_hbm, sidx_hbm, tabs_hbm, z2d_hbm, ones_hbm, acc_out,
             g_idx, s_idx, rows0, rows1, acc_sh, gs0, gs1, ss0, ss1):
    cid = lax.axis_index("c")
    sid = lax.axis_index("s")
    bufs = (rows0, rows1)
    gsems = (gs0, gs1)
    ssems = (ss0, ss1)
    myrows = pl.ds(sid * ROWS_PER_TILE, ROWS_PER_TILE)

    pltpu.sync_copy(z2d_hbm.at[myrows], acc_sh.at[myrows])
    plsc.subcore_barrier()

    @pl.when(cid == 1)
    def _():
      with jax.named_scope("deg_phase"):
        pltpu.sync_copy(ones_hbm, rows0)
        for p in range(DEG_CHUNKS // PASS_CHUNKS):
            pltpu.sync_copy(
                sidx_hbm.at[pl.ds(DEG_CHUNKS * sid + PASS_CHUNKS * p,
                                  PASS_CHUNKS)], s_idx)

            def dgroup(i, carry):
                descs = []
                for j in range(DEG_RING):
                    descs.append(pltpu.async_copy(
                        rows0, acc_sh.at[s_idx.at[DEG_RING * i + j]],
                        ssems[j % 2], add=True))
                for j in range(DEG_RING):
                    descs[j].wait()
                return carry

            lax.fori_loop(0, PASS_CHUNKS // DEG_RING, dgroup, 0)
        plsc.subcore_barrier()
        pltpu.sync_copy(acc_sh.at[myrows],
                        acc_out.at[pl.ds(N_PAD + sid * ROWS_PER_TILE,
                                         ROWS_PER_TILE)])
        pltpu.sync_copy(z2d_hbm.at[myrows], acc_sh.at[myrows])
        plsc.subcore_barrier()

    base = jnp.where(cid == 0, K0 * sid, 16 * K0 + K1 * sid)

    def gather(c, b):
        return pltpu.async_copy(tabs_hbm.at[g_idx.at[c]], bufs[b], gsems[b])

    def run_pass(p):
        row0 = base + PASS_CHUNKS * p
        pltpu.sync_copy(gidx_hbm.at[pl.ds(row0, PASS_CHUNKS)], g_idx)
        pltpu.sync_copy(sidx_hbm.at[pl.ds(row0, PASS_CHUNKS)], s_idx)
        for b in range(NBUF):
            gather(b, b)

        def group(i, carry):
            descs = []
            for b in range(NBUF):
                c = NBUF * i + b
                pltpu.make_async_copy(tabs_hbm.at[g_idx.at[c]], bufs[b],
                                      gsems[b]).wait()
                descs.append(pltpu.async_copy(
                    bufs[b], acc_sh.at[s_idx.at[c]], ssems[b], add=True))
            for b in range(NBUF):
                nxt = NBUF * i + b + NBUF

                @pl.when(nxt < PASS_CHUNKS)
                def _():
                    descs[b].wait()
                    gather(nxt, b)
            return carry

        lax.fori_loop(0, PASS_CHUNKS // NBUF, group, 0)
        for b in range(NBUF):
            pltpu.make_async_copy(
                bufs[b], acc_sh.at[s_idx.at[PASS_CHUNKS - NBUF + b]],
                ssems[b]).wait()

    with jax.named_scope("t_phase_shared"):
        for p in range(K1 // PASS_CHUNKS):
            run_pass(p)
    with jax.named_scope("t_phase_extra"):
        for p in range(K1 // PASS_CHUNKS, K0 // PASS_CHUNKS):

            @pl.when(cid == 0)
            def _():
                run_pass(p)

    plsc.subcore_barrier()
    pltpu.sync_copy(
        acc_sh.at[myrows],
        acc_out.at[pl.ds(2 * N_PAD * cid + sid * ROWS_PER_TILE,
                         ROWS_PER_TILE)])


def _sc_scatter(gidx, sidx, tabs, z2d, ones2d):
    mesh = plsc.VectorSubcoreMesh(core_axis_name="c", subcore_axis_name="s")
    f = pl.kernel(
        _sc_body,
        mesh=mesh,
        out_type=[
            jax.ShapeDtypeStruct((3 * N_PAD, HID), jnp.float32),
        ],
        scratch_types=[
            pltpu.VMEM((PASS_CHUNKS, CHUNK), jnp.int32),
            pltpu.VMEM((PASS_CHUNKS, CHUNK), jnp.int32),
            pltpu.VMEM((CHUNK, HID), jnp.float32),
            pltpu.VMEM((CHUNK, HID), jnp.float32),
            pltpu.VMEM_SHARED((N_PAD, HID), jnp.float32),
            pltpu.SemaphoreType.DMA,
            pltpu.SemaphoreType.DMA,
            pltpu.SemaphoreType.DMA,
            pltpu.SemaphoreType.DMA,
        ],
    )
    return f(gidx, sidx, tabs, z2d, ones2d)


def _post_body(h_ref, a_ref, accT0_ref, accT1_ref, deg_ref, nw1a_ref,
               nw1b_ref, nb1_ref, nw2_ref, nb2_ref, out_ref):
    hv = h_ref[...]
    nb1 = nb1_ref[...]
    nw2 = nw2_ref[...]
    nb2 = nb2_ref[...]
    mi1 = accT0_ref[...] + accT1_ref[...]
    mi0 = a_ref[...] * deg_ref[...][:, 0:1]

    def mlp(v, w1):
        return jnp.dot(jax.nn.relu(
            jnp.dot(v, w1, preferred_element_type=jnp.float32) + nb1),
            nw2, preferred_element_type=jnp.float32) + nb2

    u0 = hv + mlp(mi0, nw1a_ref[...])
    u1 = hv + mlp(mi1, nw1a_ref[...])
    u2 = hv + mlp(hv, nw1b_ref[...])
    out_ref[...] = jnp.concatenate(
        [u0[:, None, :], u1[:, None, :], u2[:, None, :]], axis=1)


def _tc_post(h2, a, accT0, accT1, deg, nw1a, nw1b, nb1, nw2, nb2, blk=1000):
    grid = N_NODES // blk
    full = lambda s: pl.BlockSpec(s, lambda i: (0,) * len(s))
    return pl.pallas_call(
        _post_body,
        grid=(grid,),
        in_specs=[
            pl.BlockSpec((blk, HID), lambda i: (i, 0)),
            pl.BlockSpec((blk, HID), lambda i: (i, 0)),
            pl.BlockSpec((blk, HID), lambda i: (i, 0)),
            pl.BlockSpec((blk, HID), lambda i: (i, 0)),
            pl.BlockSpec((blk, HID), lambda i: (i, 0)),
            full((HID, HID)), full((HID, HID)), full((1, HID)),
            full((HID, HID)), full((1, HID)),
        ],
        out_specs=pl.BlockSpec((blk, 3, HID), lambda i: (i, 0, 0)),
        out_shape=jax.ShapeDtypeStruct((N_NODES, 3, HID), jnp.float32),
    )(h2, a, accT0, accT1, deg, nw1a, nw1b, nb1, nw2, nb2)


def kernel(h, x, edge_index, edge_W1, edge_b1, edge_W2, edge_b2, inf_W,
           inf_b, node_W1, node_b1, node_W2, node_b2):
    h2 = h[:, 0, :]
    a_tab, t_tab = _tc_pre(
        h2, edge_W1[:HID], edge_W1[HID:], edge_b1[None, :], edge_W2,
        edge_b2[None, :], inf_W, inf_b[None, :])

    src = edge_index[0]
    dst = edge_index[1]
    pad = E_PAD - N_EDGES
    gidx = jnp.concatenate([src, jnp.zeros((pad,), jnp.int32)]).reshape(
        N_CHUNKS, CHUNK)
    dump = N_NODES + jnp.arange(pad, dtype=jnp.int32) % (N_PAD - N_NODES)
    sidx = jnp.concatenate([dst, dump]).reshape(
        N_CHUNKS, CHUNK)
    z2d = jnp.zeros((N_PAD, HID), jnp.float32)
    ones2d = jnp.ones((CHUNK, HID), jnp.float32)

    (acc_flat,) = _sc_scatter(gidx, sidx, t_tab, z2d, ones2d)
    accT0 = acc_flat[:N_NODES]
    deg = acc_flat[N_PAD:N_PAD + N_NODES]
    accT1 = acc_flat[2 * N_PAD:2 * N_PAD + N_NODES]

    h_new = _tc_post(h2, a_tab, accT0, accT1, deg, node_W1[:HID],
                     node_W1[HID:], node_b1[None, :], node_W2,
                     node_b2[None, :])
    return (h_new, x)


# speedup vs baseline: 2.4821x; 2.4231x over previous
import functools

import jax
import jax.numpy as jnp
from jax import lax
from jax.experimental import pallas as pl
from jax.experimental.pallas import tpu as pltpu
from jax.experimental.pallas import tpu_sc as plsc

N_NODES = 10000
HID = 128
N_EDGES = 320000

N_PAD = 10112
ROWS_PER_TILE = N_PAD // 16
CHUNK = 128
E_PAD = 327680
EDGES_PER_TILE = E_PAD // 16
CHUNKS_PER_TILE = EDGES_PER_TILE // CHUNK


def _pre_body(h_ref, w1a_ref, w1b_ref, b1_ref, w2_ref, b2_ref, iw_ref,
              ib_ref, a_ref, t_ref):
    hv = h_ref[...]
    b1 = b1_ref[...]
    w2 = w2_ref[...]
    b2 = b2_ref[...]
    iw = iw_ref[...]
    ib = ib_ref[...]
    fa = jnp.dot(jax.nn.relu(jnp.dot(hv, w1a_ref[...],
                                     preferred_element_type=jnp.float32) + b1),
                 w2, preferred_element_type=jnp.float32) + b2
    fb = jnp.dot(jax.nn.relu(jnp.dot(hv, w1b_ref[...],
                                     preferred_element_type=jnp.float32) + b1),
                 w2, preferred_element_type=jnp.float32) + b2
    sa = jax.nn.sigmoid(jnp.dot(fa, iw, preferred_element_type=jnp.float32) + ib)
    sb = jax.nn.sigmoid(jnp.dot(fb, iw, preferred_element_type=jnp.float32) + ib)
    a_ref[...] = fa * sa
    t_ref[...] = fb * sb


def _tc_pre(h2, w1a, w1b, b1, w2, b2, iw, ib, blk=1000):
    grid = N_NODES // blk
    full = lambda s: pl.BlockSpec(s, lambda i: (0,) * len(s))
    return pl.pallas_call(
        _pre_body,
        grid=(grid,),
        in_specs=[
            pl.BlockSpec((blk, HID), lambda i: (i, 0)),
            full((HID, HID)), full((HID, HID)), full((1, HID)),
            full((HID, HID)), full((1, HID)), full((HID, 1)), full((1, 1)),
        ],
        out_specs=[
            pl.BlockSpec((blk, HID), lambda i: (i, 0)),
            pl.BlockSpec((blk, HID), lambda i: (i, 0)),
        ],
        out_shape=[
            jax.ShapeDtypeStruct((N_NODES, HID), jnp.float32),
            jax.ShapeDtypeStruct((N_NODES, HID), jnp.float32),
        ],
    )(h2, w1a, w1b, b1, w2, b2, iw, ib)


NBUF = 2
PASS_CHUNKS = 40
N_CHUNKS = E_PAD // CHUNK
K0 = 120
K1 = 40
DEG_CHUNKS = N_CHUNKS // 16
DEG_RING = 8


def _sc_body(gidx_hbm, sidx_hbm, tabs_hbm, z2d_hbm, ones_hbm, acc_out,
             g_idx, s_idx, rows0, rows1, acc_sh, gs0, gs1, ss0, ss1):
    cid = lax.axis_index("c")
    sid = lax.axis_index("s")
    bufs = (rows0, rows1)
    gsems = (gs0, gs1)
    ssems = (ss0, ss1)
    myrows = pl.ds(sid * ROWS_PER_TILE, ROWS_PER_TILE)

    pltpu.sync_copy(z2d_hbm.at[myrows], acc_sh.at[myrows])
    plsc.subcore_barrier()

    @pl.when(cid == 1)
    def _():
      with jax.named_scope("deg_phase"):
        pltpu.sync_copy(ones_hbm, rows0)
        for p in range(DEG_CHUNKS // PASS_CHUNKS):
            pltpu.sync_copy(
                sidx_hbm.at[pl.ds(DEG_CHUNKS * sid + PASS_CHUNKS * p,
                                  PASS_CHUNKS)], s_idx)

            def dgroup(i, carry):
                descs = []
                for j in range(DEG_RING):
                    descs.append(pltpu.async_copy(
                        rows0, acc_sh.at[s_idx.at[DEG_RING * i + j]],
                        ssems[j % 2], add=True))
                for j in range(DEG_RING):
                    descs[j].wait()
                return carry

            lax.fori_loop(0, PASS_CHUNKS // DEG_RING, dgroup, 0)
        plsc.subcore_barrier()
        pltpu.sync_copy(acc_sh.at[myrows],
                        acc_out.at[pl.ds(N_PAD + sid * ROWS_PER_TILE,
                                         ROWS_PER_TILE)])
        pltpu.sync_copy(z2d_hbm.at[myrows], acc_sh.at[myrows])
        plsc.subcore_barrier()

    base = jnp.where(cid == 0, K0 * sid, 16 * K0 + K1 * sid)

    def gather(c, b):
        return pltpu.async_copy(tabs_hbm.at[g_idx.at[c]], bufs[b], gsems[b])

    def run_pass(p):
        row0 = base + PASS_CHUNKS * p
        pltpu.sync_copy(gidx_hbm.at[pl.ds(row0, PASS_CHUNKS)], g_idx)
        pltpu.sync_copy(sidx_hbm.at[pl.ds(row0, PASS_CHUNKS)], s_idx)
        for b in range(NBUF):
            gather(b, b)

        def group(i, carry):
            descs = []
            for b in range(NBUF):
                c = NBUF * i + b
                pltpu.make_async_copy(tabs_hbm.at[g_idx.at[c]], bufs[b],
                                      gsems[b]).wait()
                descs.append(pltpu.async_copy(
                    bufs[b], acc_sh.at[s_idx.at[c]], ssems[b], add=True))
            for b in range(NBUF):
                nxt = NBUF * i + b + NBUF

                @pl.when(nxt < PASS_CHUNKS)
                def _():
                    descs[b].wait()
                    gather(nxt, b)
            return carry

        lax.fori_loop(0, PASS_CHUNKS // NBUF, group, 0)
        for b in range(NBUF):
            pltpu.make_async_copy(
                bufs[b], acc_sh.at[s_idx.at[PASS_CHUNKS - NBUF + b]],
                ssems[b]).wait()

    with jax.named_scope("t_phase_shared"):
        for p in range(K1 // PASS_CHUNKS):
            run_pass(p)
    with jax.named_scope("t_phase_extra"):
        for p in range(K1 // PASS_CHUNKS, K0 // PASS_CHUNKS):

            @pl.when(cid == 0)
            def _():
                run_pass(p)

    plsc.subcore_barrier()
    pltpu.sync_copy(
        acc_sh.at[myrows],
        acc_out.at[pl.ds(2 * N_PAD * cid + sid * ROWS_PER_TILE,
                         ROWS_PER_TILE)])


def _sc_scatter(gidx, sidx, tabs, z2d, ones2d):
    mesh = plsc.VectorSubcoreMesh(core_axis_name="c", subcore_axis_name="s")
    f = pl.kernel(
        _sc_body,
        mesh=mesh,
        out_type=[
            jax.ShapeDtypeStruct((3 * N_PAD, HID), jnp.float32),
        ],
        scratch_types=[
            pltpu.VMEM((PASS_CHUNKS, CHUNK), jnp.int32),
            pltpu.VMEM((PASS_CHUNKS, CHUNK), jnp.int32),
            pltpu.VMEM((CHUNK, HID), jnp.float32),
            pltpu.VMEM((CHUNK, HID), jnp.float32),
            pltpu.VMEM_SHARED((N_PAD, HID), jnp.float32),
            pltpu.SemaphoreType.DMA,
            pltpu.SemaphoreType.DMA,
            pltpu.SemaphoreType.DMA,
            pltpu.SemaphoreType.DMA,
        ],
    )
    return f(gidx, sidx, tabs, z2d, ones2d)


def _post_body(h_ref, a_ref, accT0_ref, accT1_ref, deg_ref, nw1a_ref,
               nw1b_ref, nb1_ref, nw2_ref, nb2_ref, out_ref):
    hv = h_ref[...]
    nb1 = nb1_ref[...]
    nw2 = nw2_ref[...]
    nb2 = nb2_ref[...]
    mi1 = accT0_ref[...] + accT1_ref[...]
    mi0 = a_ref[...] * deg_ref[...][:, 0:1]

    def mlp(v, w1):
        return jnp.dot(jax.nn.relu(
            jnp.dot(v, w1, preferred_element_type=jnp.float32) + nb1),
            nw2, preferred_element_type=jnp.float32) + nb2

    u0 = hv + mlp(mi0, nw1a_ref[...])
    u1 = hv + mlp(mi1, nw1a_ref[...])
    u2 = hv + mlp(hv, nw1b_ref[...])
    out_ref[...] = jnp.concatenate(
        [u0[:, None, :], u1[:, None, :], u2[:, None, :]], axis=1)


def _tc_post(h2, a, accT0, accT1, deg, nw1a, nw1b, nb1, nw2, nb2, blk=1000):
    grid = N_NODES // blk
    full = lambda s: pl.BlockSpec(s, lambda i: (0,) * len(s))
    return pl.pallas_call(
        _post_body,
        grid=(grid,),
        in_specs=[
            pl.BlockSpec((blk, HID), lambda i: (i, 0)),
            pl.BlockSpec((blk, HID), lambda i: (i, 0)),
            pl.BlockSpec((blk, HID), lambda i: (i, 0)),
            pl.BlockSpec((blk, HID), lambda i: (i, 0)),
            pl.BlockSpec((blk, HID), lambda i: (i, 0)),
            full((HID, HID)), full((HID, HID)), full((1, HID)),
            full((HID, HID)), full((1, HID)),
        ],
        out_specs=pl.BlockSpec((blk, 3, HID), lambda i: (i, 0, 0)),
        out_shape=jax.ShapeDtypeStruct((N_NODES, 3, HID), jnp.float32),
    )(h2, a, accT0, accT1, deg, nw1a, nw1b, nb1, nw2, nb2)


def kernel(h, x, edge_index, edge_W1, edge_b1, edge_W2, edge_b2, inf_W,
           inf_b, node_W1, node_b1, node_W2, node_b2):
    h2 = h[:, 0, :]
    a_tab, t_tab = _tc_pre(
        h2, edge_W1[:HID], edge_W1[HID:], edge_b1[None, :], edge_W2,
        edge_b2[None, :], inf_W, inf_b[None, :])

    src = edge_index[0]
    dst = edge_index[1]
    pad = E_PAD - N_EDGES
    gpad = jnp.arange(pad, dtype=jnp.int32) % N_NODES
    gidx = jnp.concatenate([src, gpad]).reshape(
        N_CHUNKS, CHUNK)
    dump = N_NODES + jnp.arange(pad, dtype=jnp.int32) % (N_PAD - N_NODES)
    sidx = jnp.concatenate([dst, dump]).reshape(
        N_CHUNKS, CHUNK)
    z2d = jnp.zeros((N_PAD, HID), jnp.float32)
    ones2d = jnp.ones((CHUNK, HID), jnp.float32)

    (acc_flat,) = _sc_scatter(gidx, sidx, t_tab, z2d, ones2d)
    accT0 = acc_flat[:N_NODES]
    deg = acc_flat[N_PAD:N_PAD + N_NODES]
    accT1 = acc_flat[2 * N_PAD:2 * N_PAD + N_NODES]

    h_new = _tc_post(h2, a_tab, accT0, accT1, deg, node_W1[:HID],
                     node_W1[HID:], node_b1[None, :], node_W2,
                     node_b2[None, :])
    return (h_new, x)
